# Initial kernel scaffold; baseline (speedup 1.0000x reference)
import jax
import jax.numpy as jnp
from jax.experimental import pallas as pl


def kernel(x, p_grid):
    raise NotImplementedError("write your pallas kernel here")


# speedup vs baseline: 27.6845x; 27.6845x over previous
import functools

import jax
import jax.numpy as jnp
from jax import lax
from jax.experimental import pallas as pl

N2 = 8192
K = 10
R2V = 0.0625
Q = 128
C = 1024
NCHUNK = N2 // C


def _cumsum_lanes(a, width):
    k = 1
    rows = a.shape[0]
    while k < width:
        shifted = jnp.concatenate(
            [jnp.zeros((rows, k), a.dtype), a[:, : width - k]], axis=1)
        a = a + shifted
        k *= 2
    return a


def _bq_body(q_ref, xt_ref, w_ref, map_ref, out_ref):
    qx = q_ref[:, 0:1]
    qy = q_ref[:, 1:2]
    qz = q_ref[:, 2:3]

    def chunk_step(c, carry):
        def compute(carry):
            count, accs = carry
            cx = xt_ref[0:1, pl.ds(c * C, C)]
            cy = xt_ref[1:2, pl.ds(c * C, C)]
            cz = xt_ref[2:3, pl.ds(c * C, C)]
            dx = qx - cx
            dy = qy - cy
            dz = qz - cz
            d2 = dx * dx + dy * dy + dz * dz
            within = d2 <= R2V
            wi = within.astype(jnp.int32)
            excl = count + (_cumsum_lanes(wi, C) - wi)
            wchunk = w_ref[pl.ds(c * C, C), :]
            new_accs = []
            for s in range(K):
                mask_f = jnp.where(within & (excl == s), 1.0, 0.0)
                r = lax.dot_general(
                    mask_f, wchunk, (((1,), (0,)), ((), ())),
                    precision=lax.Precision.HIGHEST,
                    preferred_element_type=jnp.float32)
                new_accs.append(accs[s] + r)
            new_count = count + jnp.sum(wi, axis=1, keepdims=True)
            return new_count, tuple(new_accs)

        count, _ = carry
        done = jnp.min(count) >= K
        return lax.cond(done, lambda cr: cr, compute, carry)

    count0 = jnp.zeros((Q, 1), jnp.int32)
    accs0 = tuple(jnp.zeros((Q, 4), jnp.float32) for _ in range(K))
    _, accs = lax.fori_loop(0, NCHUNK, chunk_step, (count0, accs0))

    map_ref[...] = jnp.concatenate(
        [accs[s][:, 3:4] for s in range(K)], axis=1).astype(jnp.int32)
    out_ref[...] = jnp.concatenate(
        [accs[s][:, 0:3].reshape(Q, 1, 3) for s in range(K)], axis=1)


@jax.jit
def kernel(x, p_grid):
    b = x.shape[0]
    x2 = x[0]
    xt = x2.T
    idx_col = jnp.arange(N2, dtype=jnp.float32)[:, None]
    w = jnp.concatenate([x2, idx_col], axis=1)
    p_flat = p_grid.reshape(N2, 3)

    grid = (N2 // Q,)
    mapping, outputs = pl.pallas_call(
        _bq_body,
        grid=grid,
        in_specs=[
            pl.BlockSpec((Q, 3), lambda i: (i, 0)),
            pl.BlockSpec((3, N2), lambda i: (0, 0)),
            pl.BlockSpec((N2, 4), lambda i: (0, 0)),
        ],
        out_specs=[
            pl.BlockSpec((Q, K), lambda i: (i, 0)),
            pl.BlockSpec((Q, K, 3), lambda i: (i, 0, 0)),
        ],
        out_shape=[
            jax.ShapeDtypeStruct((N2, K), jnp.int32),
            jax.ShapeDtypeStruct((N2, K, 3), jnp.float32),
        ],
    )(p_flat, xt, w)

    return mapping.reshape(b, N2, K), outputs.reshape(b, N2, K, 3)


# speedup vs baseline: 100.3599x; 3.6251x over previous
import functools

import jax
import jax.numpy as jnp
from jax import lax
from jax.experimental import pallas as pl
from jax.experimental.pallas import tpu as pltpu
from jax.experimental.pallas import tpu_sc as plsc

N2 = 8192
K = 10
R2V = 0.0625
L = 16
NW = 32
QPW = N2 // NW
ROW = 16


def _sc_body(xs_h, ys_h, zs_h, qx_h, qy_h, qz_h,
             map_h, ox_h, oy_h, oz_h,
             xs, ys, zs, qx, qy, qz, mbuf, oxb, oyb, ozb):
    wid = lax.axis_index("s") * 2 + lax.axis_index("c")
    base = wid * QPW

    pltpu.sync_copy(xs_h, xs)
    pltpu.sync_copy(ys_h, ys)
    pltpu.sync_copy(zs_h, zs)
    pltpu.sync_copy(qx_h.at[pl.ds(base, QPW)], qx)
    pltpu.sync_copy(qy_h.at[pl.ds(base, QPW)], qy)
    pltpu.sync_copy(qz_h.at[pl.ds(base, QPW)], qz)

    zi = jnp.zeros((L,), jnp.int32)
    zf = jnp.zeros((L,), jnp.float32)

    def zero_body(i, _):
        mbuf[pl.ds(i * L, L)] = zi
        oxb[pl.ds(i * L, L)] = zf
        oyb[pl.ds(i * L, L)] = zf
        ozb[pl.ds(i * L, L)] = zf
        return 0

    lax.fori_loop(0, QPW, zero_body, 0)

    iota = lax.iota(jnp.int32, L)

    def qblock_body(qb, _):
        qvx = qx[pl.ds(qb * L, L)]
        qvy = qy[pl.ds(qb * L, L)]
        qvz = qz[pl.ds(qb * L, L)]
        for i in range(L):
            q = qb * L + i
            qxs = qvx[i]
            qys = qvy[i]
            qzs = qvz[i]

            def cond(carry):
                j, cnt = carry
                return jnp.logical_and(j < N2, cnt < K)

            def step(carry, qxs=qxs, qys=qys, qzs=qzs, q=q):
                j, cnt = carry
                cx = xs[pl.ds(j, L)]
                cy = ys[pl.ds(j, L)]
                cz = zs[pl.ds(j, L)]
                dx = cx - qxs
                dy = cy - qys
                dz = cz - qzs
                d2 = dx * dx + dy * dy + dz * dz
                within = d2 <= R2V
                wi = within.astype(jnp.int32)
                excl = plsc.cumsum(wi) - wi
                slot = excl + cnt
                valid = jnp.logical_and(within, slot < K)
                fidx = q * ROW + slot
                plsc.store_scatter(mbuf, [fidx], j + iota, mask=valid)
                plsc.store_scatter(oxb, [fidx], cx, mask=valid)
                plsc.store_scatter(oyb, [fidx], cy, mask=valid)
                plsc.store_scatter(ozb, [fidx], cz, mask=valid)
                return (j + L, cnt + jnp.sum(wi))

            lax.while_loop(cond, step, (jnp.int32(0), jnp.int32(0)))
        return 0

    lax.fori_loop(0, QPW // L, qblock_body, 0)

    pltpu.sync_copy(mbuf, map_h.at[pl.ds(base * ROW, QPW * ROW)])
    pltpu.sync_copy(oxb, ox_h.at[pl.ds(base * ROW, QPW * ROW)])
    pltpu.sync_copy(oyb, oy_h.at[pl.ds(base * ROW, QPW * ROW)])
    pltpu.sync_copy(ozb, oz_h.at[pl.ds(base * ROW, QPW * ROW)])


_sc_ball_query = functools.partial(
    pl.kernel,
    out_type=[
        jax.ShapeDtypeStruct((N2 * ROW,), jnp.int32),
        jax.ShapeDtypeStruct((N2 * ROW,), jnp.float32),
        jax.ShapeDtypeStruct((N2 * ROW,), jnp.float32),
        jax.ShapeDtypeStruct((N2 * ROW,), jnp.float32),
    ],
    mesh=plsc.VectorSubcoreMesh(core_axis_name="c", subcore_axis_name="s"),
    compiler_params=pltpu.CompilerParams(needs_layout_passes=False),
    scratch_types=[
        pltpu.VMEM((N2,), jnp.float32),
        pltpu.VMEM((N2,), jnp.float32),
        pltpu.VMEM((N2,), jnp.float32),
        pltpu.VMEM((QPW,), jnp.float32),
        pltpu.VMEM((QPW,), jnp.float32),
        pltpu.VMEM((QPW,), jnp.float32),
        pltpu.VMEM((QPW * ROW,), jnp.int32),
        pltpu.VMEM((QPW * ROW,), jnp.float32),
        pltpu.VMEM((QPW * ROW,), jnp.float32),
        pltpu.VMEM((QPW * ROW,), jnp.float32),
    ],
)(_sc_body)


@jax.jit
def kernel(x, p_grid):
    b = x.shape[0]
    x2 = x[0]
    p2 = p_grid.reshape(N2, 3)
    m, ox, oy, oz = _sc_ball_query(
        x2[:, 0], x2[:, 1], x2[:, 2], p2[:, 0], p2[:, 1], p2[:, 2])
    mapping = m.reshape(N2, ROW)[:, :K]
    outputs = jnp.stack(
        [ox.reshape(N2, ROW)[:, :K],
         oy.reshape(N2, ROW)[:, :K],
         oz.reshape(N2, ROW)[:, :K]], axis=-1)
    return mapping.reshape(b, N2, K), outputs.reshape(b, N2, K, 3)


# speedup vs baseline: 104.1142x; 1.0374x over previous
import functools

import jax
import jax.numpy as jnp
from jax import lax
from jax.experimental import pallas as pl
from jax.experimental.pallas import tpu as pltpu
from jax.experimental.pallas import tpu_sc as plsc

N2 = 8192
K = 10
R2V = 0.0625
L = 16
NW = 32
QPW = N2 // NW
ROW = 16
G = 4


def _sc_body(xs_h, ys_h, zs_h, qx_h, qy_h, qz_h,
             map_h, ox_h, oy_h, oz_h,
             xs, ys, zs, qx, qy, qz, mbuf, oxb, oyb, ozb):
    wid = lax.axis_index("s") * 2 + lax.axis_index("c")
    base = wid * QPW

    pltpu.sync_copy(xs_h, xs)
    pltpu.sync_copy(ys_h, ys)
    pltpu.sync_copy(zs_h, zs)
    pltpu.sync_copy(qx_h.at[pl.ds(base, QPW)], qx)
    pltpu.sync_copy(qy_h.at[pl.ds(base, QPW)], qy)
    pltpu.sync_copy(qz_h.at[pl.ds(base, QPW)], qz)

    zi = jnp.zeros((L,), jnp.int32)
    zf = jnp.zeros((L,), jnp.float32)

    def zero_body(i, _):
        mbuf[pl.ds(i * L, L)] = zi
        oxb[pl.ds(i * L, L)] = zf
        oyb[pl.ds(i * L, L)] = zf
        ozb[pl.ds(i * L, L)] = zf
        return 0

    lax.fori_loop(0, QPW, zero_body, 0)

    iota = lax.iota(jnp.int32, L)

    def qblock_body(qb, _):
        qvx = qx[pl.ds(qb * L, L)]
        qvy = qy[pl.ds(qb * L, L)]
        qvz = qz[pl.ds(qb * L, L)]
        for i in range(L):
            q = qb * L + i
            qxs = qvx[i]
            qys = qvy[i]
            qzs = qvz[i]

            def cond(carry):
                j, cnt = carry
                return jnp.logical_and(j < N2, cnt < K)

            def step(carry, qxs=qxs, qys=qys, qzs=qzs, q=q):
                j, cnt = carry
                qrow = q * ROW
                tot = cnt
                for g in range(G):
                    off = j + g * L
                    cx = xs[pl.ds(off, L)]
                    cy = ys[pl.ds(off, L)]
                    cz = zs[pl.ds(off, L)]
                    dx = cx - qxs
                    dy = cy - qys
                    dz = cz - qzs
                    d2 = dx * dx + dy * dy + dz * dz
                    within = d2 <= R2V
                    wi = within.astype(jnp.int32)
                    excl = plsc.cumsum(wi) - wi
                    n_g = plsc.all_reduce_population_count(within)
                    slot = excl + tot
                    valid = jnp.logical_and(within, slot < K)
                    fidx = qrow + slot
                    plsc.store_scatter(mbuf, [fidx], off + iota, mask=valid)
                    plsc.store_scatter(oxb, [fidx], cx, mask=valid)
                    plsc.store_scatter(oyb, [fidx], cy, mask=valid)
                    plsc.store_scatter(ozb, [fidx], cz, mask=valid)
                    tot = tot + n_g
                return (j + G * L, tot[0])

            lax.while_loop(cond, step, (jnp.int32(0), jnp.int32(0)))
        return 0

    lax.fori_loop(0, QPW // L, qblock_body, 0)

    pltpu.sync_copy(mbuf, map_h.at[pl.ds(base * ROW, QPW * ROW)])
    pltpu.sync_copy(oxb, ox_h.at[pl.ds(base * ROW, QPW * ROW)])
    pltpu.sync_copy(oyb, oy_h.at[pl.ds(base * ROW, QPW * ROW)])
    pltpu.sync_copy(ozb, oz_h.at[pl.ds(base * ROW, QPW * ROW)])


_sc_ball_query = functools.partial(
    pl.kernel,
    out_type=[
        jax.ShapeDtypeStruct((N2 * ROW,), jnp.int32),
        jax.ShapeDtypeStruct((N2 * ROW,), jnp.float32),
        jax.ShapeDtypeStruct((N2 * ROW,), jnp.float32),
        jax.ShapeDtypeStruct((N2 * ROW,), jnp.float32),
    ],
    mesh=plsc.VectorSubcoreMesh(core_axis_name="c", subcore_axis_name="s"),
    compiler_params=pltpu.CompilerParams(needs_layout_passes=False),
    scratch_types=[
        pltpu.VMEM((N2,), jnp.float32),
        pltpu.VMEM((N2,), jnp.float32),
        pltpu.VMEM((N2,), jnp.float32),
        pltpu.VMEM((QPW,), jnp.float32),
        pltpu.VMEM((QPW,), jnp.float32),
        pltpu.VMEM((QPW,), jnp.float32),
        pltpu.VMEM((QPW * ROW,), jnp.int32),
        pltpu.VMEM((QPW * ROW,), jnp.float32),
        pltpu.VMEM((QPW * ROW,), jnp.float32),
        pltpu.VMEM((QPW * ROW,), jnp.float32),
    ],
)(_sc_body)


@jax.jit
def kernel(x, p_grid):
    b = x.shape[0]
    x2 = x[0]
    p2 = p_grid.reshape(N2, 3)
    m, ox, oy, oz = _sc_ball_query(
        x2[:, 0], x2[:, 1], x2[:, 2], p2[:, 0], p2[:, 1], p2[:, 2])
    mapping = m.reshape(N2, ROW)[:, :K]
    outputs = jnp.stack(
        [ox.reshape(N2, ROW)[:, :K],
         oy.reshape(N2, ROW)[:, :K],
         oz.reshape(N2, ROW)[:, :K]], axis=-1)
    return mapping.reshape(b, N2, K), outputs.reshape(b, N2, K, 3)


# speedup vs baseline: 157.2449x; 1.5103x over previous
import functools

import jax
import jax.numpy as jnp
from jax import lax
from jax.experimental import pallas as pl
from jax.experimental.pallas import tpu as pltpu
from jax.experimental.pallas import tpu_sc as plsc

N2 = 8192
K = 10
R2V = 0.0625
L = 16
NW = 32
QPW = N2 // NW
ROW = 16
B = 4


def _sc_body(xs_h, ys_h, zs_h, qx_h, qy_h, qz_h,
             map_h, ox_h, oy_h, oz_h,
             xs, ys, zs, qx, qy, qz, mbuf, oxb, oyb, ozb):
    wid = lax.axis_index("s") * 2 + lax.axis_index("c")
    base = wid * QPW

    pltpu.sync_copy(xs_h, xs)
    pltpu.sync_copy(ys_h, ys)
    pltpu.sync_copy(zs_h, zs)
    pltpu.sync_copy(qx_h.at[pl.ds(base, QPW)], qx)
    pltpu.sync_copy(qy_h.at[pl.ds(base, QPW)], qy)
    pltpu.sync_copy(qz_h.at[pl.ds(base, QPW)], qz)

    zi = jnp.zeros((L,), jnp.int32)
    zf = jnp.zeros((L,), jnp.float32)

    def zero_body(i, _):
        mbuf[pl.ds(i * L, L)] = zi
        oxb[pl.ds(i * L, L)] = zf
        oyb[pl.ds(i * L, L)] = zf
        ozb[pl.ds(i * L, L)] = zf
        return 0

    lax.fori_loop(0, QPW, zero_body, 0)

    iota = lax.iota(jnp.int32, L)

    def qblock_body(qb, _):
        qvx = qx[pl.ds(qb * L, L)]
        qvy = qy[pl.ds(qb * L, L)]
        qvz = qz[pl.ds(qb * L, L)]
        for batch in range(L // B):
            qs = [(qvx[batch * B + b], qvy[batch * B + b], qvz[batch * B + b])
                  for b in range(B)]

            def cond(carry):
                j = carry[0]
                cnts = carry[1:]
                not_done = cnts[0] < K
                for c in cnts[1:]:
                    not_done = jnp.logical_or(not_done, c < K)
                return jnp.logical_and(j < N2, not_done)

            def step(carry, qs=qs, batch=batch):
                j = carry[0]
                cnts = list(carry[1:])
                cx = xs[pl.ds(j, L)]
                cy = ys[pl.ds(j, L)]
                cz = zs[pl.ds(j, L)]
                cand = j + iota
                for b in range(B):
                    qxs, qys, qzs = qs[b]
                    q = qb * L + batch * B + b
                    dx = cx - qxs
                    dy = cy - qys
                    dz = cz - qzs
                    d2 = dx * dx + dy * dy + dz * dz
                    within = d2 <= R2V
                    wi = within.astype(jnp.int32)
                    excl = plsc.cumsum(wi) - wi
                    n_b = plsc.all_reduce_population_count(within)
                    slot = excl + cnts[b]
                    valid = jnp.logical_and(within, slot < K)
                    fidx = q * ROW + slot
                    plsc.store_scatter(mbuf, [fidx], cand, mask=valid)
                    plsc.store_scatter(oxb, [fidx], cx, mask=valid)
                    plsc.store_scatter(oyb, [fidx], cy, mask=valid)
                    plsc.store_scatter(ozb, [fidx], cz, mask=valid)
                    cnts[b] = cnts[b] + n_b[0]
                return (j + L, *cnts)

            lax.while_loop(cond, step,
                           (jnp.int32(0),) + (jnp.int32(0),) * B)
        return 0

    lax.fori_loop(0, QPW // L, qblock_body, 0)

    pltpu.sync_copy(mbuf, map_h.at[pl.ds(base * ROW, QPW * ROW)])
    pltpu.sync_copy(oxb, ox_h.at[pl.ds(base * ROW, QPW * ROW)])
    pltpu.sync_copy(oyb, oy_h.at[pl.ds(base * ROW, QPW * ROW)])
    pltpu.sync_copy(ozb, oz_h.at[pl.ds(base * ROW, QPW * ROW)])


_sc_ball_query = functools.partial(
    pl.kernel,
    out_type=[
        jax.ShapeDtypeStruct((N2 * ROW,), jnp.int32),
        jax.ShapeDtypeStruct((N2 * ROW,), jnp.float32),
        jax.ShapeDtypeStruct((N2 * ROW,), jnp.float32),
        jax.ShapeDtypeStruct((N2 * ROW,), jnp.float32),
    ],
    mesh=plsc.VectorSubcoreMesh(core_axis_name="c", subcore_axis_name="s"),
    compiler_params=pltpu.CompilerParams(needs_layout_passes=False),
    scratch_types=[
        pltpu.VMEM((N2,), jnp.float32),
        pltpu.VMEM((N2,), jnp.float32),
        pltpu.VMEM((N2,), jnp.float32),
        pltpu.VMEM((QPW,), jnp.float32),
        pltpu.VMEM((QPW,), jnp.float32),
        pltpu.VMEM((QPW,), jnp.float32),
        pltpu.VMEM((QPW * ROW,), jnp.int32),
        pltpu.VMEM((QPW * ROW,), jnp.float32),
        pltpu.VMEM((QPW * ROW,), jnp.float32),
        pltpu.VMEM((QPW * ROW,), jnp.float32),
    ],
)(_sc_body)


@jax.jit
def kernel(x, p_grid):
    b = x.shape[0]
    x2 = x[0]
    p2 = p_grid.reshape(N2, 3)
    m, ox, oy, oz = _sc_ball_query(
        x2[:, 0], x2[:, 1], x2[:, 2], p2[:, 0], p2[:, 1], p2[:, 2])
    mapping = m.reshape(N2, ROW)[:, :K]
    outputs = jnp.stack(
        [ox.reshape(N2, ROW)[:, :K],
         oy.reshape(N2, ROW)[:, :K],
         oz.reshape(N2, ROW)[:, :K]], axis=-1)
    return mapping.reshape(b, N2, K), outputs.reshape(b, N2, K, 3)
